# baseline (speedup 1.0000x reference)
import dataclasses
import functools

import jax
import jax.numpy as jnp
from jax import lax
from jax.experimental import pallas as pl
from jax.experimental.pallas import tpu as pltpu
from jax.experimental.pallas import tpu_sc as plsc

N_NODES = 10000
N_EDGES = 320000
N_GRAPHS = 256

NC, NS = 2, 16
NW = NC * NS
N_PAD = 10240
NPT = N_PAD // NW
EB = 64
NB = 160
EPW = NB * EB
NE_PAD = EPW * NW
EPT = NE_PAD // NS
NROW = N_PAD // 16
G_PAD = 264

_MESH = plsc.VectorSubcoreMesh(
    core_axis_name="c", subcore_axis_name="s", num_cores=NC, num_subcores=NS
)
_CP = dataclasses.replace(
    pltpu.CompilerParams(), use_tc_tiling_on_sc=False, needs_layout_passes=False
)
_F32 = jnp.float32
_HIGHEST = jax.lax.Precision.HIGHEST


def _rsqrt_newton(x):
    i = plsc.bitcast(x, jnp.int32)
    y = plsc.bitcast(jnp.int32(0x5F3759DF) - (i >> 1), _F32)
    for _ in range(3):
        y = y * (1.5 - 0.5 * x * y * y)
    return y


def _degree_scales(src_e, dst_e, iden):
    @functools.partial(
        pl.kernel,
        out_type=(jax.ShapeDtypeStruct((NROW, 16), _F32),
                  jax.ShapeDtypeStruct((NROW, 16), _F32)),
        mesh=_MESH,
        scratch_types=[
            pltpu.VMEM((EPT,), jnp.int32),
            pltpu.VMEM((NROW, 16), _F32),
            pltpu.VMEM((40, 16), _F32),
            pltpu.VMEM((40, 16), _F32),
            pltpu.VMEM((5, 128), jnp.int32),
            pltpu.VMEM_SHARED((NROW, 16), _F32),
        ],
        compiler_params=_CP,
    )
    def k(src_hbm, dst_hbm, iden_hbm, so_hbm, si_hbm, e_v, hist_v, buf_v,
          s_v, iden_v, hist_sh):
        cid = lax.axis_index("c")
        sid = lax.axis_index("s")
        zeros16 = jnp.zeros((16,), _F32)
        ones16 = jnp.ones((16,), _F32)

        def work(edge_hbm, out_hbm):
            pltpu.sync_copy(edge_hbm.at[sid], e_v)
            pltpu.sync_copy(iden_hbm, iden_v)

            @pl.loop(0, NROW)
            def _(r):
                hist_v[r, :] = zeros16

            @pl.loop(0, 40)
            def _(r):
                buf_v[r, :] = zeros16

            @pl.loop(0, EPT // 16)
            def _(i):
                idx = e_v[pl.ds(i * 16, 16)]
                plsc.addupdate_scatter(hist_v, [idx >> 4, idx & 15], ones16)

            pltpu.sync_copy(buf_v, hist_sh.at[pl.ds(sid * 40, 40)])
            plsc.subcore_barrier()
            for j in range(5):
                pltpu.sync_copy(hist_v.at[pl.ds(j * 128, 128)],
                                hist_sh.at[iden_v.at[j]], add=True)
            plsc.subcore_barrier()

            pltpu.sync_copy(hist_sh.at[pl.ds(sid * 40, 40)], buf_v)

            @pl.loop(0, 40)
            def _(r):
                x = jnp.maximum(buf_v[r, :], 1.0)
                s_v[r, :] = _rsqrt_newton(x)

            pltpu.sync_copy(s_v, out_hbm.at[pl.ds(sid * 40, 40)])

        @pl.when(cid == 0)
        def _():
            work(src_hbm, so_hbm)

        @pl.when(cid == 1)
        def _():
            work(dst_hbm, si_hbm)

    return k(src_e, dst_e, iden)


def _embed_scale(emb, text, s_out):
    @functools.partial(
        pl.kernel,
        out_type=jax.ShapeDtypeStruct((N_PAD, 64), _F32),
        mesh=_MESH,
        scratch_types=[
            pltpu.VMEM((NPT,), jnp.int32),
            pltpu.VMEM((NPT, 64), _F32),
            pltpu.VMEM((NPT // 16, 16), _F32),
            pltpu.SemaphoreType.DMA,
        ],
        compiler_params=_CP,
    )
    def k(emb_hbm, text_hbm, s_hbm, out_hbm, txt_v, rows_v, s_v, sem):
        cid = lax.axis_index("c")
        sid = lax.axis_index("s")
        wid = cid * NS + sid
        base = wid * NPT
        pltpu.sync_copy(text_hbm.at[pl.ds(base, NPT)], txt_v)
        pltpu.sync_copy(s_hbm.at[pl.ds(wid * (NPT // 16), NPT // 16)], s_v)
        for b in range(NPT // 64):
            pltpu.async_copy(emb_hbm.at[txt_v.at[pl.ds(b * 64, 64)]],
                             rows_v.at[pl.ds(b * 64, 64)], sem).wait()

        @pl.loop(0, NPT // 16)
        def _(r):
            v = s_v[r, :]
            for j in range(16):
                n = r * 16 + j
                sc = v[j]
                for c in range(4):
                    rows_v[n, pl.ds(c * 16, 16)] = rows_v[n, pl.ds(c * 16, 16)] * sc

        pltpu.sync_copy(rows_v, out_hbm.at[pl.ds(base, NPT)])

    return k(emb, text, s_out)


def _seg_sum(src, dst, h):
    D = h.shape[1]
    rows_per_tile = N_PAD // NS

    @functools.partial(
        pl.kernel,
        out_type=jax.ShapeDtypeStruct((NC, N_PAD, D), _F32),
        mesh=_MESH,
        scratch_types=[
            pltpu.VMEM((NB, EB), jnp.int32),
            pltpu.VMEM((NB, EB), jnp.int32),
            pltpu.VMEM((EB, D), _F32),
            pltpu.VMEM((EB, D), _F32),
            pltpu.VMEM_SHARED((N_PAD, D), _F32),
            pltpu.SemaphoreType.DMA,
            pltpu.SemaphoreType.DMA,
        ],
        compiler_params=_CP,
    )
    def k(src_hbm, dst_hbm, h_hbm, out_hbm, src_v, dst_v, rows0_v, rows1_v,
          acc_sh, sem0, sem1):
        cid = lax.axis_index("c")
        sid = lax.axis_index("s")
        wid = cid * NS + sid

        pltpu.sync_copy(src_hbm.at[wid], src_v)
        pltpu.sync_copy(dst_hbm.at[wid], dst_v)

        zeros16 = jnp.zeros((16,), _F32)

        @pl.loop(0, EB)
        def _(r):
            @pl.loop(0, D, step=16)
            def _(cc):
                rows0_v[r, pl.ds(cc, 16)] = zeros16

        @pl.loop(0, rows_per_tile, step=EB)
        def _(j):
            pltpu.sync_copy(rows0_v, acc_sh.at[pl.ds(sid * rows_per_tile + j, EB)])

        pltpu.async_copy(h_hbm.at[src_v.at[0]], rows0_v, sem0)
        pltpu.async_copy(h_hbm.at[src_v.at[1]], rows1_v, sem1)
        plsc.subcore_barrier()

        @pl.loop(0, NB, step=2)
        def _(b):
            pltpu.make_async_copy(h_hbm.at[src_v.at[0]], rows0_v, sem0).wait()
            pltpu.sync_copy(rows0_v, acc_sh.at[dst_v.at[b]], add=True)

            @pl.when(b + 2 < NB)
            def _():
                pltpu.async_copy(h_hbm.at[src_v.at[b + 2]], rows0_v, sem0)

            pltpu.make_async_copy(h_hbm.at[src_v.at[1]], rows1_v, sem1).wait()
            pltpu.sync_copy(rows1_v, acc_sh.at[dst_v.at[b + 1]], add=True)

            @pl.when(b + 3 < NB)
            def _():
                pltpu.async_copy(h_hbm.at[src_v.at[b + 3]], rows1_v, sem1)

        plsc.subcore_barrier()
        r0 = sid * rows_per_tile
        pltpu.sync_copy(acc_sh.at[pl.ds(r0, rows_per_tile)],
                        out_hbm.at[cid, pl.ds(r0, rows_per_tile)])

    return k(src, dst, h)


def _mm(parts, s_in, W, b, s_out=None):
    N, D = parts.shape[1], parts.shape[2]
    F = W.shape[1]
    nblk = N // 256
    specs = [
        pl.BlockSpec((2, 256, D), lambda i: (0, i, 0)),
        pl.BlockSpec((256, 1), lambda i: (i, 0)),
        pl.BlockSpec((D, F), lambda i: (0, 0)),
        pl.BlockSpec((1, F), lambda i: (0, 0)),
    ]
    args = [parts, s_in, W, b.reshape(1, F)]
    if s_out is not None:
        specs.append(pl.BlockSpec((256, 1), lambda i: (i, 0)))
        args.append(s_out)

    def body(p_ref, si_ref, w_ref, b_ref, *rest):
        o_ref = rest[-1]
        x = (p_ref[0] + p_ref[1]) * si_ref[...]
        y = lax.dot_general(x, w_ref[...], (((1,), (0,)), ((), ())),
                            precision=_HIGHEST, preferred_element_type=_F32)
        y = jnp.maximum(y + b_ref[...], 0.0)
        if s_out is not None:
            y = y * rest[0][...]
        o_ref[...] = y

    return pl.pallas_call(
        body, grid=(nblk,), in_specs=specs,
        out_specs=pl.BlockSpec((256, F), lambda i: (i, 0)),
        out_shape=jax.ShapeDtypeStruct((N, F), _F32),
    )(*args)


def _seg_max_partials(h2, gids):
    @functools.partial(
        pl.kernel,
        out_type=jax.ShapeDtypeStruct((NW, G_PAD, 128), _F32),
        mesh=_MESH,
        scratch_types=[
            pltpu.VMEM((NPT, 128), _F32),
            pltpu.VMEM((NPT,), jnp.int32),
            pltpu.VMEM((G_PAD, 128), _F32),
        ],
        compiler_params=_CP,
    )
    def k(h_hbm, g_hbm, out_hbm, rows_v, gid_v, part_v):
        cid = lax.axis_index("c")
        sid = lax.axis_index("s")
        wid = cid * NS + sid
        base = wid * NPT
        pltpu.sync_copy(h_hbm.at[pl.ds(base, NPT)], rows_v)
        pltpu.sync_copy(g_hbm.at[pl.ds(base, NPT)], gid_v)
        zeros16 = jnp.zeros((16,), _F32)

        @pl.loop(0, G_PAD)
        def _(r):
            @pl.loop(0, 128, step=16)
            def _(cc):
                part_v[r, pl.ds(cc, 16)] = zeros16

        @pl.loop(0, NPT // 16)
        def _(r):
            v = gid_v[pl.ds(r * 16, 16)]
            for j in range(16):
                n = r * 16 + j
                g = v[j]
                for c in range(8):
                    cur = part_v[g, pl.ds(c * 16, 16)]
                    x = rows_v[n, pl.ds(c * 16, 16)]
                    part_v[g, pl.ds(c * 16, 16)] = jnp.maximum(cur, x)

        pltpu.sync_copy(part_v, out_hbm.at[wid])

    return k(h2, gids)


def _readout(partials, Wc, bc):
    F = Wc.shape[1]

    def body(p_ref, w_ref, b_ref, o_ref, acc_ref):
        i = pl.program_id(0)
        x = p_ref[0, :N_GRAPHS, :]

        @pl.when(i == 0)
        def _():
            acc_ref[...] = x

        @pl.when(i > 0)
        def _():
            acc_ref[...] = jnp.maximum(acc_ref[...], x)

        @pl.when(i == NW - 1)
        def _():
            o_ref[...] = lax.dot_general(
                acc_ref[...], w_ref[...], (((1,), (0,)), ((), ())),
                precision=_HIGHEST, preferred_element_type=_F32) + b_ref[...]

    return pl.pallas_call(
        body, grid=(NW,),
        in_specs=[
            pl.BlockSpec((1, G_PAD, 128), lambda i: (i, 0, 0)),
            pl.BlockSpec((128, F), lambda i: (0, 0)),
            pl.BlockSpec((1, F), lambda i: (0, 0)),
        ],
        out_specs=pl.BlockSpec((N_GRAPHS, F), lambda i: (0, 0)),
        out_shape=jax.ShapeDtypeStruct((N_GRAPHS, F), _F32),
        scratch_shapes=[pltpu.VMEM((N_GRAPHS, 128), _F32)],
    )(partials, Wc, bc.reshape(1, F))


def kernel(text, edge_index, graph_ids, emb, W1, b1, W2, b2, Wc, bc):
    text = text.astype(jnp.int32)
    graph_ids = graph_ids.astype(jnp.int32)
    src = edge_index[0].astype(jnp.int32)
    dst = edge_index[1].astype(jnp.int32)

    pad_e = NE_PAD - N_EDGES
    src_p = jnp.concatenate([src, jnp.full((pad_e,), N_NODES, jnp.int32)])
    dst_p = jnp.concatenate([dst, jnp.full((pad_e,), N_NODES, jnp.int32)])
    text_p = jnp.concatenate(
        [text, jnp.zeros((N_PAD - N_NODES,), jnp.int32)])
    gids_p = jnp.concatenate(
        [graph_ids, jnp.full((N_PAD - N_NODES,), N_GRAPHS, jnp.int32)])
    iden = jnp.arange(NROW, dtype=jnp.int32).reshape(5, 128)

    s_out_g, s_in_g = _degree_scales(
        src_p.reshape(NS, EPT), dst_p.reshape(NS, EPT), iden)
    s_out_col = s_out_g.reshape(N_PAD, 1)
    s_in_col = s_in_g.reshape(N_PAD, 1)

    h0s = _embed_scale(emb, text_p, s_out_g)

    src_w = src_p.reshape(NW, NB, EB)
    dst_w = dst_p.reshape(NW, NB, EB)

    parts1 = _seg_sum(src_w, dst_w, h0s)
    h1s = _mm(parts1, s_in_col, W1, b1, s_out=s_out_col)
    parts2 = _seg_sum(src_w, dst_w, h1s)
    h2 = _mm(parts2, s_in_col, W2, b2)
    partials = _seg_max_partials(h2, gids_p)
    return _readout(partials, Wc, bc)
